# Initial kernel scaffold; baseline (speedup 1.0000x reference)
import jax
import jax.numpy as jnp
from jax.experimental import pallas as pl


def kernel(time_vector, linear_trend, constant_offset, seasonal_amplitudes, seasonal_phases, longterm_amplitudes, longterm_phases, longterm_periods, spatial_weights, neighbor_weights, neighbor_indices):
    raise NotImplementedError("write your pallas kernel here")


# speedup vs baseline: 69.4941x; 69.4941x over previous
import math

import jax
import jax.numpy as jnp
from jax import lax
from jax.experimental import pallas as pl
from jax.experimental.pallas import tpu as pltpu
from jax.experimental.pallas import tpu_sc as plsc

K = 16
TBL = 16
T = 128
NW = 32
SB = 320
G = 16
PBN = 2048
BN = 1024

_SEASONAL_FREQS = (4.0, 2.0, 1.0)


def _prep_body(arr_ref, out_ref):
    x = arr_ref[...]
    c = lax.broadcasted_iota(jnp.int32, x.shape, 1)
    out_ref[...] = jnp.where(
        c < 3, x, jnp.where(c < 6, jnp.cos(x), jnp.where(c < 9, jnp.sin(x), 0.0)))


def _sc_smooth_body(table_hbm, idx_hbm, w_hbm, out_hbm,
                    idx_sb, w_sb, rows_v, out_sb, sem):
    n_pad = out_hbm.shape[0]
    s_w = n_pad // NW
    nsb = s_w // SB
    ng = SB // G
    wid = lax.axis_index("s") * 2 + lax.axis_index("c")
    lane = lax.iota(jnp.int32, 16)

    def sb_body(sb, carry):
        base = pl.multiple_of(wid * s_w + sb * SB, SB)
        pltpu.sync_copy(
            idx_hbm.at[pl.ds(pl.multiple_of(base * K // 128, SB * K // 128),
                             SB * K // 128)], idx_sb)
        pltpu.sync_copy(w_hbm.at[pl.ds(pl.multiple_of(base * K, SB * K),
                                       SB * K)], w_sb)

        def g_body(t, carry2):
            cp0 = pltpu.async_copy(table_hbm.at[idx_sb.at[2 * t]],
                                   rows_v.at[0], sem)
            cp1 = pltpu.async_copy(table_hbm.at[idx_sb.at[2 * t + 1]],
                                   rows_v.at[1], sem)
            cp0.wait()
            cp1.wait()
            gb = t * (G * K)
            accs = [jnp.zeros((16,), jnp.float32) for _ in range(9)]
            for k in range(K):
                r = lane * K + k
                wk = plsc.load_gather(w_sb, [gb + r])
                hi = r >> 7
                lo = r & 127
                for c in range(9):
                    col = jnp.full((16,), c, jnp.int32)
                    v = plsc.load_gather(rows_v, [hi, lo, col])
                    accs[c] = accs[c] + wk * v
            srow = t * G + lane
            for c in range(9):
                col = jnp.full((16,), c, jnp.int32)
                plsc.store_scatter(out_sb, [srow, col], accs[c])
            return carry2

        lax.fori_loop(0, ng, g_body, 0)
        pltpu.sync_copy(out_sb, out_hbm.at[pl.ds(base, SB)])
        return carry

    lax.fori_loop(0, nsb, sb_body, 0)


def _final_body(te_ref, tbl_ref, wavg_ref, tr_ref, off_ref, sw_ref,
                la_ref, lp_ref, pd_ref, out_ref):
    te = te_ref[...]
    tbl = tbl_ref[...]
    wavg = wavg_ref[...]
    sw = sw_ref[...]
    mix = 1.0 / (1.0 + jnp.exp(-sw))
    sm = (1.0 - mix) * tbl + mix * wavg
    sig = off_ref[...] + tr_ref[...] * te
    two_pi = 2.0 * math.pi
    for i, f in enumerate(_SEASONAL_FREQS):
        amp = sm[:, i:i + 1]
        mr = sm[:, 3 + i:4 + i]
        mi = sm[:, 6 + i:7 + i]
        h = jnp.sqrt(mr * mr + mi * mi)
        inv = amp / jnp.maximum(h, 1e-30)
        arg = (two_pi * f) * te
        sig = sig + (inv * mr) * jnp.sin(arg) + (inv * mi) * jnp.cos(arg)
    la = la_ref[...]
    lp = lp_ref[...]
    for j in range(2):
        aj = la[:, j:j + 1]
        pj = lp[:, j:j + 1]
        arg = (two_pi * te) / pd_ref[:, j:j + 1]
        sig = sig + (aj * jnp.cos(pj)) * jnp.sin(arg) \
                  + (aj * jnp.sin(pj)) * jnp.cos(arg)
    out_ref[...] = sig


def kernel(time_vector, linear_trend, constant_offset, seasonal_amplitudes,
           seasonal_phases, longterm_amplitudes, longterm_phases,
           longterm_periods, spatial_weights, neighbor_weights,
           neighbor_indices):
    f32 = jnp.float32
    n = linear_trend.shape[0]
    n_pad = -(-n // (NW * SB)) * (NW * SB)
    pad = n_pad - n

    arr = jnp.concatenate(
        [seasonal_amplitudes, seasonal_phases, seasonal_phases,
         jnp.zeros((n, TBL - 9), f32)], axis=1)
    arr = jnp.pad(arr, ((0, pad), (0, 0)))
    idx2d = jnp.pad(neighbor_indices, ((0, pad), (0, 0))).reshape(
        n_pad * K // 128, 128)
    w_flat = jnp.pad(neighbor_weights, ((0, pad), (0, 0))).reshape(-1)

    table = pl.pallas_call(
        _prep_body,
        grid=(n_pad // PBN,),
        in_specs=[pl.BlockSpec((PBN, TBL), lambda i: (i, 0))],
        out_specs=pl.BlockSpec((PBN, TBL), lambda i: (i, 0)),
        out_shape=jax.ShapeDtypeStruct((n_pad, TBL), f32),
    )(arr)

    wavg = pl.kernel(
        _sc_smooth_body,
        out_type=jax.ShapeDtypeStruct((n_pad, TBL), f32),
        mesh=plsc.VectorSubcoreMesh(core_axis_name="c", subcore_axis_name="s",
                                    num_cores=2, num_subcores=16),
        compiler_params=pltpu.CompilerParams(needs_layout_passes=False,
                                             use_tc_tiling_on_sc=False),
        scratch_types=[
            pltpu.VMEM((SB * K // 128, 128), jnp.int32),
            pltpu.VMEM((SB * K,), f32),
            pltpu.VMEM((2, 128, TBL), f32),
            pltpu.VMEM((SB, TBL), f32),
            pltpu.SemaphoreType.DMA,
        ],
    )(table, idx2d, w_flat)

    te2 = time_vector.reshape(1, T)
    tr2 = jnp.pad(linear_trend, (0, pad)).reshape(n_pad, 1)
    off2 = jnp.pad(constant_offset, (0, pad)).reshape(n_pad, 1)
    sw2 = jnp.pad(spatial_weights, (0, pad)).reshape(n_pad, 1)
    la2 = jnp.pad(longterm_amplitudes, ((0, pad), (0, 0)))
    lp2 = jnp.pad(longterm_phases, ((0, pad), (0, 0)))
    pd2 = longterm_periods.reshape(1, 2)
    out = pl.pallas_call(
        _final_body,
        grid=(n_pad // BN,),
        in_specs=[
            pl.BlockSpec((1, T), lambda i: (0, 0)),
            pl.BlockSpec((BN, TBL), lambda i: (i, 0)),
            pl.BlockSpec((BN, TBL), lambda i: (i, 0)),
            pl.BlockSpec((BN, 1), lambda i: (i, 0)),
            pl.BlockSpec((BN, 1), lambda i: (i, 0)),
            pl.BlockSpec((BN, 1), lambda i: (i, 0)),
            pl.BlockSpec((BN, 2), lambda i: (i, 0)),
            pl.BlockSpec((BN, 2), lambda i: (i, 0)),
            pl.BlockSpec((1, 2), lambda i: (0, 0)),
        ],
        out_specs=pl.BlockSpec((BN, T), lambda i: (i, 0)),
        out_shape=jax.ShapeDtypeStruct((n_pad, T), f32),
    )(te2, table, wavg, tr2, off2, sw2, la2, lp2, pd2)
    return out[:n]


# speedup vs baseline: 80.8727x; 1.1637x over previous
import math

import jax
import jax.numpy as jnp
from jax import lax
from jax.experimental import pallas as pl
from jax.experimental.pallas import tpu as pltpu
from jax.experimental.pallas import tpu_sc as plsc

K = 16
TBL = 16
T = 128
NW = 32
SB = 640
G = 16
PBN = 2048
BN = 1024

_SEASONAL_FREQS = (4.0, 2.0, 1.0)


def _prep_body(arr_ref, out_ref):
    x = arr_ref[...]
    c = lax.broadcasted_iota(jnp.int32, x.shape, 1)
    out_ref[...] = jnp.where(
        c < 3, x, jnp.where(c < 6, jnp.cos(x), jnp.where(c < 9, jnp.sin(x), 0.0)))


def _sc_smooth_body(table_hbm, idx_hbm, w_hbm, out_hbm,
                    idx_sb, w_sb, rows_v, out_sb, sem):
    n_pad = out_hbm.shape[0]
    s_w = n_pad // NW
    nsb = s_w // SB
    ng = SB // G
    nrow = SB * K // 128
    wid = lax.axis_index("s") * 2 + lax.axis_index("c")
    lane = lax.iota(jnp.int32, 16)
    hi = lane >> 3
    lane16 = (lane & 7) * K

    def stage(sb, buf):
        base = pl.multiple_of(wid * s_w + sb * SB, SB)
        ci = pltpu.async_copy(
            idx_hbm.at[pl.ds(pl.multiple_of(base * K // 128, nrow), nrow)],
            idx_sb.at[buf], sem.at[8])
        cw = pltpu.async_copy(
            w_hbm.at[pl.ds(pl.multiple_of(base * K, SB * K), SB * K)],
            w_sb.at[buf], sem.at[9])
        return ci, cw

    def wait_stage(buf):
        pltpu.make_async_copy(idx_hbm.at[pl.ds(0, nrow)], idx_sb.at[buf],
                              sem.at[8]).wait()
        pltpu.make_async_copy(w_hbm.at[pl.ds(0, SB * K)], w_sb.at[buf],
                              sem.at[9]).wait()

    def fire(cur, j, slot):
        pltpu.async_copy(table_hbm.at[idx_sb.at[cur, j]], rows_v.at[slot],
                         sem.at[slot])

    def wait_slot(slot):
        pltpu.make_async_copy(table_hbm.at[pl.ds(0, 128)], rows_v.at[slot],
                              sem.at[slot]).wait()

    stage(0, 0)
    wait_stage(0)

    def sb_body(sb, carry):
        cur = lax.rem(sb, 2)
        base = pl.multiple_of(wid * s_w + sb * SB, SB)

        @pl.when(sb + 1 < nsb)
        def _():
            stage(sb + 1, 1 - cur)

        for j in range(8):
            fire(cur, j, j)

        def round_body(r, carry2):
            for gi in range(4):
                t = 4 * r + gi
                s0, s1 = 2 * gi, 2 * gi + 1
                wait_slot(s0)
                wait_slot(s1)
                gb = t * (G * K)
                accs = [jnp.zeros((16,), jnp.float32) for _ in range(9)]
                for k in range(K):
                    wk = plsc.load_gather(w_sb.at[cur], [gb + lane * K + k])
                    lo = lane16 + k
                    bsel = s0 + hi
                    for c in range(9):
                        col = jnp.full((16,), c, jnp.int32)
                        v = plsc.load_gather(rows_v, [bsel, lo, col])
                        accs[c] = accs[c] + wk * v
                srow = t * G + lane
                for c in range(9):
                    col = jnp.full((16,), c, jnp.int32)
                    plsc.store_scatter(out_sb, [srow, col], accs[c])

                @pl.when(t + 4 < ng)
                def _():
                    fire(cur, 2 * t + 8, s0)
                    fire(cur, 2 * t + 9, s1)
            return carry2

        lax.fori_loop(0, ng // 4, round_body, 0)
        pltpu.sync_copy(out_sb, out_hbm.at[pl.ds(base, SB)])

        @pl.when(sb + 1 < nsb)
        def _():
            wait_stage(1 - cur)
        return carry

    lax.fori_loop(0, nsb, sb_body, 0)


def _final_body(te_ref, tbl_ref, wavg_ref, tr_ref, off_ref, sw_ref,
                la_ref, lp_ref, pd_ref, out_ref):
    te = te_ref[...]
    tbl = tbl_ref[...]
    wavg = wavg_ref[...]
    sw = sw_ref[...]
    mix = 1.0 / (1.0 + jnp.exp(-sw))
    sm = (1.0 - mix) * tbl + mix * wavg
    sig = off_ref[...] + tr_ref[...] * te
    two_pi = 2.0 * math.pi
    for i, f in enumerate(_SEASONAL_FREQS):
        amp = sm[:, i:i + 1]
        mr = sm[:, 3 + i:4 + i]
        mi = sm[:, 6 + i:7 + i]
        h = jnp.sqrt(mr * mr + mi * mi)
        inv = amp / jnp.maximum(h, 1e-30)
        arg = (two_pi * f) * te
        sig = sig + (inv * mr) * jnp.sin(arg) + (inv * mi) * jnp.cos(arg)
    la = la_ref[...]
    lp = lp_ref[...]
    for j in range(2):
        aj = la[:, j:j + 1]
        pj = lp[:, j:j + 1]
        arg = (two_pi * te) / pd_ref[:, j:j + 1]
        sig = sig + (aj * jnp.cos(pj)) * jnp.sin(arg) \
                  + (aj * jnp.sin(pj)) * jnp.cos(arg)
    out_ref[...] = sig


def kernel(time_vector, linear_trend, constant_offset, seasonal_amplitudes,
           seasonal_phases, longterm_amplitudes, longterm_phases,
           longterm_periods, spatial_weights, neighbor_weights,
           neighbor_indices):
    f32 = jnp.float32
    n = linear_trend.shape[0]
    n_pad = -(-n // (NW * SB)) * (NW * SB)
    pad = n_pad - n

    arr = jnp.concatenate(
        [seasonal_amplitudes, seasonal_phases, seasonal_phases,
         jnp.zeros((n, TBL - 9), f32)], axis=1)
    arr = jnp.pad(arr, ((0, pad), (0, 0)))
    idx2d = jnp.pad(neighbor_indices, ((0, pad), (0, 0))).reshape(
        n_pad * K // 128, 128)
    w_flat = jnp.pad(neighbor_weights, ((0, pad), (0, 0))).reshape(-1)

    table = pl.pallas_call(
        _prep_body,
        grid=(n_pad // PBN,),
        in_specs=[pl.BlockSpec((PBN, TBL), lambda i: (i, 0))],
        out_specs=pl.BlockSpec((PBN, TBL), lambda i: (i, 0)),
        out_shape=jax.ShapeDtypeStruct((n_pad, TBL), f32),
    )(arr)

    wavg = pl.kernel(
        _sc_smooth_body,
        out_type=jax.ShapeDtypeStruct((n_pad, TBL), f32),
        mesh=plsc.VectorSubcoreMesh(core_axis_name="c", subcore_axis_name="s",
                                    num_cores=2, num_subcores=16),
        compiler_params=pltpu.CompilerParams(needs_layout_passes=False,
                                             use_tc_tiling_on_sc=False),
        scratch_types=[
            pltpu.VMEM((2, SB * K // 128, 128), jnp.int32),
            pltpu.VMEM((2, SB * K), f32),
            pltpu.VMEM((8, 128, TBL), f32),
            pltpu.VMEM((SB, TBL), f32),
            pltpu.SemaphoreType.DMA((10,)),
        ],
    )(table, idx2d, w_flat)

    te2 = time_vector.reshape(1, T)
    tr2 = jnp.pad(linear_trend, (0, pad)).reshape(n_pad, 1)
    off2 = jnp.pad(constant_offset, (0, pad)).reshape(n_pad, 1)
    sw2 = jnp.pad(spatial_weights, (0, pad)).reshape(n_pad, 1)
    la2 = jnp.pad(longterm_amplitudes, ((0, pad), (0, 0)))
    lp2 = jnp.pad(longterm_phases, ((0, pad), (0, 0)))
    pd2 = longterm_periods.reshape(1, 2)
    out = pl.pallas_call(
        _final_body,
        grid=(n_pad // BN,),
        in_specs=[
            pl.BlockSpec((1, T), lambda i: (0, 0)),
            pl.BlockSpec((BN, TBL), lambda i: (i, 0)),
            pl.BlockSpec((BN, TBL), lambda i: (i, 0)),
            pl.BlockSpec((BN, 1), lambda i: (i, 0)),
            pl.BlockSpec((BN, 1), lambda i: (i, 0)),
            pl.BlockSpec((BN, 1), lambda i: (i, 0)),
            pl.BlockSpec((BN, 2), lambda i: (i, 0)),
            pl.BlockSpec((BN, 2), lambda i: (i, 0)),
            pl.BlockSpec((1, 2), lambda i: (0, 0)),
        ],
        out_specs=pl.BlockSpec((BN, T), lambda i: (i, 0)),
        out_shape=jax.ShapeDtypeStruct((n_pad, T), f32),
    )(te2, table, wavg, tr2, off2, sw2, la2, lp2, pd2)
    return out[:n]


# speedup vs baseline: 164.4332x; 2.0332x over previous
import math

import jax
import jax.numpy as jnp
from jax import lax
from jax.experimental import pallas as pl
from jax.experimental.pallas import tpu as pltpu
from jax.experimental.pallas import tpu_sc as plsc

K = 16
TBL = 16
T = 128
NW = 32
SB = 640
G = 16
BN = 2048

_SEASONAL_FREQS = (4.0, 2.0, 1.0)
_TWO_PI = 2.0 * math.pi


def _dot(a, b):
    return jnp.dot(a.astype(jnp.bfloat16), b.astype(jnp.bfloat16),
                   preferred_element_type=jnp.float32)


def _prep_body(av_ref, out_ref):
    xw = av_ref[...]
    c = lax.broadcasted_iota(jnp.int32, xw.shape, 1) % 16
    out_ref[...] = jnp.where(c < 3, xw,
                             jnp.where(c < 6, jnp.cos(xw),
                                       jnp.where(c < 9, jnp.sin(xw), 0.0)))


def _prep_lt_body(av3_ref, out_ref):
    xw = av3_ref[...]
    c = lax.broadcasted_iota(jnp.int32, xw.shape, 1) % 16
    out_ref[...] = jnp.where(c < 2, jnp.cos(xw),
                             jnp.where(c < 4, jnp.sin(xw), 0.0))


def _sc_smooth_body(table_hbm, idx_hbm, w_hbm, sw_hbm,
                    outa_hbm, outr_hbm, outi_hbm,
                    idx_sb, w_sb, self_sb, sw_sb, idx_flat, rows_v, osb, sem):
    n = outa_hbm.shape[0]
    s_w = (n + NW - 1) // NW
    s_w = s_w + (SB - s_w % SB) % SB
    nsb = s_w // SB
    ng = SB // G
    clamp = n - SB
    wid = lax.axis_index("s") * 2 + lax.axis_index("c")
    lane = lax.iota(jnp.int32, 16)
    hi = lane >> 3

    def stage(sb, buf):
        base = pl.multiple_of(
            jnp.minimum(wid * s_w + sb * SB, clamp), 8)
        pltpu.async_copy(idx_hbm.at[pl.ds(base, SB)], idx_sb.at[buf],
                         sem.at[8])
        pltpu.async_copy(w_hbm.at[pl.ds(base, SB)], w_sb.at[buf],
                         sem.at[9])
        pltpu.async_copy(table_hbm.at[pl.ds(base, SB)], self_sb.at[buf],
                         sem.at[10])
        pltpu.async_copy(sw_hbm.at[pl.ds(base, SB)], sw_sb.at[buf],
                         sem.at[11])

    def wait_stage(buf):
        pltpu.make_async_copy(idx_hbm.at[pl.ds(0, SB)], idx_sb.at[buf],
                              sem.at[8]).wait()
        pltpu.make_async_copy(w_hbm.at[pl.ds(0, SB)], w_sb.at[buf],
                              sem.at[9]).wait()
        pltpu.make_async_copy(table_hbm.at[pl.ds(0, SB)], self_sb.at[buf],
                              sem.at[10]).wait()
        pltpu.make_async_copy(sw_hbm.at[pl.ds(0, SB)], sw_sb.at[buf],
                              sem.at[11]).wait()

    def fire(half, slot):
        pltpu.async_copy(table_hbm.at[idx_flat.at[pl.ds(half * 128, 128)]],
                         rows_v.at[slot], sem.at[slot])

    def wait_slot(slot):
        pltpu.make_async_copy(table_hbm.at[pl.ds(0, 128)], rows_v.at[slot],
                              sem.at[slot]).wait()

    stage(0, 0)
    wait_stage(0)

    def sb_body(sb, carry):
        cur = lax.rem(sb, 2)
        base = pl.multiple_of(
            jnp.minimum(wid * s_w + sb * SB, clamp), 8)

        @pl.when(sb + 1 < nsb)
        def _():
            stage(sb + 1, 1 - cur)

        def repack_body(s, carry2):
            idx_flat[pl.ds(s * K, K)] = idx_sb[cur, s]
            return carry2

        lax.fori_loop(0, SB, repack_body, 0)

        for j in range(8):
            fire(j, j)

        def round_body(r, carry2):
            for gi in range(4):
                t = 4 * r + gi
                s0, s1 = 2 * gi, 2 * gi + 1
                wait_slot(s0)
                wait_slot(s1)
                srow = t * G + lane
                curv = jnp.full((16,), cur, jnp.int32)
                accs = [jnp.zeros((16,), jnp.float32) for _ in range(9)]
                for k in range(K):
                    wk = plsc.load_gather(
                        w_sb, [curv, srow, jnp.full((16,), k, jnp.int32)])
                    lo = (lane & 7) * K + k
                    bsel = s0 + hi
                    for c in range(9):
                        col = jnp.full((16,), c, jnp.int32)
                        v = plsc.load_gather(rows_v, [bsel, lo, col])
                        accs[c] = accs[c] + wk * v
                swv = plsc.load_gather(sw_sb, [curv, srow])
                mix = 1.0 / (1.0 + jnp.exp(-swv))
                om = 1.0 - mix
                for c in range(9):
                    col = jnp.full((16,), c, jnp.int32)
                    sv = plsc.load_gather(self_sb, [curv, srow, col])
                    val = om * sv + mix * accs[c]
                    dst = jnp.full((16,), c // 3, jnp.int32)
                    dcol = jnp.full((16,), c % 3, jnp.int32)
                    plsc.store_scatter(osb, [dst, srow, dcol], val)

                @pl.when(t + 4 < ng)
                def _():
                    fire(2 * t + 8, s0)
                    fire(2 * t + 9, s1)
            return carry2

        lax.fori_loop(0, ng // 4, round_body, 0)
        pltpu.sync_copy(osb.at[0], outa_hbm.at[pl.ds(base, SB)])
        pltpu.sync_copy(osb.at[1], outr_hbm.at[pl.ds(base, SB)])
        pltpu.sync_copy(osb.at[2], outi_hbm.at[pl.ds(base, SB)])

        @pl.when(sb + 1 < nsb)
        def _():
            wait_stage(1 - cur)
        return carry

    lax.fori_loop(0, nsb, sb_body, 0)


def _final_body(te_ref, av2_ref, ltt_ref, wa_ref, wr_ref, wi_ref, pd_ref,
                out_ref):
    te = te_ref[...]
    av2 = av2_ref[...]
    ltt = ltt_ref[...]
    cidx = lax.broadcasted_iota(jnp.int32, av2.shape, 1)
    seas = cidx < 3
    wa = jnp.where(seas, wa_ref[...], 0.0)
    wr = jnp.where(seas, wr_ref[...], 0.0)
    wi = jnp.where(seas, wi_ref[...], 0.0)

    h2 = wr * wr + wi * wi
    inv = wa * lax.rsqrt(jnp.maximum(h2, 1e-30))
    a16 = inv * wr
    b16 = inv * wi
    ltv = av2 * ltt

    rr = lax.broadcasted_iota(jnp.int32, (16, T), 0)
    zero = jnp.zeros((16, T), jnp.float32)
    b_sin = zero
    b_cos = zero
    for i, f in enumerate(_SEASONAL_FREQS):
        arg = (_TWO_PI * f) * te
        b_sin = jnp.where(rr == i, jnp.sin(arg), b_sin)
        b_cos = jnp.where(rr == i, jnp.cos(arg), b_cos)
    b_lt = zero
    pdv = pd_ref[...]
    for j in range(2):
        arg = (_TWO_PI * te) / pdv[:, j:j + 1]
        b_lt = jnp.where(rr == j, jnp.sin(arg), b_lt)
        b_lt = jnp.where(rr == 2 + j, jnp.cos(arg), b_lt)
    b_poly = jnp.where(rr == 4, 1.0, jnp.where(rr == 5, te, 0.0))

    out_ref[...] = (_dot(av2, b_poly) + _dot(a16, b_sin) + _dot(b16, b_cos)
                    + _dot(ltv, b_lt))


def kernel(time_vector, linear_trend, constant_offset, seasonal_amplitudes,
           seasonal_phases, longterm_amplitudes, longterm_phases,
           longterm_periods, spatial_weights, neighbor_weights,
           neighbor_indices):
    f32 = jnp.float32
    n = linear_trend.shape[0]
    grid_n = (n + BN - 1) // BN

    av = jnp.concatenate(
        [seasonal_amplitudes, seasonal_phases, seasonal_phases,
         jnp.zeros((n, 7), f32)], axis=1)
    av2 = jnp.concatenate(
        [longterm_amplitudes, longterm_amplitudes,
         constant_offset[:, None], linear_trend[:, None],
         jnp.zeros((n, 10), f32)], axis=1)
    av3 = jnp.concatenate(
        [longterm_phases, longterm_phases, jnp.zeros((n, 12), f32)], axis=1)

    nw_rows = n * TBL // 128
    grid_p = (nw_rows + BN - 1) // BN
    wide_spec = dict(
        grid=(grid_p,),
        in_specs=[pl.BlockSpec((BN, 128), lambda i: (i, 0))],
        out_specs=pl.BlockSpec((BN, 128), lambda i: (i, 0)),
        out_shape=jax.ShapeDtypeStruct((nw_rows, 128), f32),
    )
    table = pl.pallas_call(_prep_body, **wide_spec)(
        av.reshape(nw_rows, 128)).reshape(n, TBL)
    ltt = pl.pallas_call(_prep_lt_body, **wide_spec)(
        av3.reshape(nw_rows, 128)).reshape(n, TBL)

    sds = jax.ShapeDtypeStruct((n, TBL), f32)
    wa, wr, wi = pl.kernel(
        _sc_smooth_body,
        out_type=(sds, sds, sds),
        mesh=plsc.VectorSubcoreMesh(core_axis_name="c", subcore_axis_name="s",
                                    num_cores=2, num_subcores=16),
        compiler_params=pltpu.CompilerParams(needs_layout_passes=False,
                                             use_tc_tiling_on_sc=False),
        scratch_types=[
            pltpu.VMEM((2, SB, K), jnp.int32),
            pltpu.VMEM((2, SB, K), f32),
            pltpu.VMEM((2, SB, TBL), f32),
            pltpu.VMEM((2, SB), f32),
            pltpu.VMEM((SB * K,), jnp.int32),
            pltpu.VMEM((8, 128, TBL), f32),
            pltpu.VMEM((3, SB, TBL), f32),
            pltpu.SemaphoreType.DMA((12,)),
        ],
    )(table, neighbor_indices, neighbor_weights, spatial_weights)

    te2 = time_vector.reshape(1, T)
    pd2 = longterm_periods.reshape(1, 2)
    out = pl.pallas_call(
        _final_body,
        grid=(grid_n,),
        in_specs=[
            pl.BlockSpec((1, T), lambda i: (0, 0)),
            pl.BlockSpec((BN, TBL), lambda i: (i, 0)),
            pl.BlockSpec((BN, TBL), lambda i: (i, 0)),
            pl.BlockSpec((BN, TBL), lambda i: (i, 0)),
            pl.BlockSpec((BN, TBL), lambda i: (i, 0)),
            pl.BlockSpec((BN, TBL), lambda i: (i, 0)),
            pl.BlockSpec((1, 2), lambda i: (0, 0)),
        ],
        out_specs=pl.BlockSpec((BN, T), lambda i: (i, 0)),
        out_shape=jax.ShapeDtypeStruct((n, T), f32),
    )(te2, av2, ltt, wa, wr, wi, pd2)
    return out


# speedup vs baseline: 165.0679x; 1.0039x over previous
import math

import jax
import jax.numpy as jnp
from jax import lax
from jax.experimental import pallas as pl
from jax.experimental.pallas import tpu as pltpu
from jax.experimental.pallas import tpu_sc as plsc

K = 16
TBL = 16
T = 128
NW = 32
SB = 640
G = 16
BN = 2048

_SEASONAL_FREQS = (4.0, 2.0, 1.0)
_TWO_PI = 2.0 * math.pi


def _dot(a, b):
    return jnp.dot(a.astype(jnp.bfloat16), b.astype(jnp.bfloat16),
                   preferred_element_type=jnp.float32)


def _prep_body(av_ref, out_ref):
    xw = av_ref[...]
    c = lax.broadcasted_iota(jnp.int32, xw.shape, 1) % 16
    out_ref[...] = jnp.where(c < 3, xw,
                             jnp.where(c < 6, jnp.cos(xw),
                                       jnp.where(c < 9, jnp.sin(xw), 0.0)))


def _prep_lt_body(av3_ref, out_ref):
    xw = av3_ref[...]
    c = lax.broadcasted_iota(jnp.int32, xw.shape, 1) % 16
    out_ref[...] = jnp.where(c < 2, jnp.cos(xw),
                             jnp.where(c < 4, jnp.sin(xw), 0.0))


def _sc_smooth_body(table_hbm, idx_hbm, w_hbm, sw_hbm, out_hbm,
                    idx_sb, w_sb, self_sb, sw_sb, idx_flat, rows_v, osb, sem):
    n = out_hbm.shape[1]
    s_w = (n + NW - 1) // NW
    s_w = s_w + (SB - s_w % SB) % SB
    nsb = s_w // SB
    ng = SB // G
    clamp = n - SB
    wid = lax.axis_index("s") * 2 + lax.axis_index("c")
    lane = lax.iota(jnp.int32, 16)
    hi = lane >> 3

    def stage(sb, buf):
        base = pl.multiple_of(
            jnp.minimum(wid * s_w + sb * SB, clamp), 8)
        pltpu.async_copy(idx_hbm.at[pl.ds(base, SB)], idx_sb.at[buf],
                         sem.at[8])
        pltpu.async_copy(w_hbm.at[pl.ds(base, SB)], w_sb.at[buf],
                         sem.at[9])
        pltpu.async_copy(table_hbm.at[pl.ds(base, SB)], self_sb.at[buf],
                         sem.at[10])
        pltpu.async_copy(sw_hbm.at[pl.ds(base, SB)], sw_sb.at[buf],
                         sem.at[11])

    def wait_stage(buf):
        pltpu.make_async_copy(idx_hbm.at[pl.ds(0, SB)], idx_sb.at[buf],
                              sem.at[8]).wait()
        pltpu.make_async_copy(w_hbm.at[pl.ds(0, SB)], w_sb.at[buf],
                              sem.at[9]).wait()
        pltpu.make_async_copy(table_hbm.at[pl.ds(0, SB)], self_sb.at[buf],
                              sem.at[10]).wait()
        pltpu.make_async_copy(sw_hbm.at[pl.ds(0, SB)], sw_sb.at[buf],
                              sem.at[11]).wait()

    def fire(half, slot):
        pltpu.async_copy(table_hbm.at[idx_flat.at[pl.ds(half * 128, 128)]],
                         rows_v.at[slot], sem.at[slot])

    def wait_slot(slot):
        pltpu.make_async_copy(table_hbm.at[pl.ds(0, 128)], rows_v.at[slot],
                              sem.at[slot]).wait()

    stage(0, 0)
    wait_stage(0)

    def sb_body(sb, carry):
        cur = lax.rem(sb, 2)
        base = pl.multiple_of(
            jnp.minimum(wid * s_w + sb * SB, clamp), 8)

        @pl.when(sb + 1 < nsb)
        def _():
            stage(sb + 1, 1 - cur)

        def repack_body(s, carry2):
            idx_flat[pl.ds(s * K, K)] = idx_sb[cur, s]
            return carry2

        lax.fori_loop(0, SB, repack_body, 0)

        for j in range(8):
            fire(j, j)

        def round_body(r, carry2):
            for gi in range(4):
                t = 4 * r + gi
                s0, s1 = 2 * gi, 2 * gi + 1
                wait_slot(s0)
                wait_slot(s1)
                srow = t * G + lane
                curv = jnp.full((16,), cur, jnp.int32)
                accs = [jnp.zeros((16,), jnp.float32) for _ in range(9)]
                for k in range(K):
                    wk = plsc.load_gather(
                        w_sb, [curv, srow, jnp.full((16,), k, jnp.int32)])
                    lo = (lane & 7) * K + k
                    bsel = s0 + hi
                    for c in range(9):
                        col = jnp.full((16,), c, jnp.int32)
                        v = plsc.load_gather(rows_v, [bsel, lo, col])
                        accs[c] = accs[c] + wk * v
                swv = plsc.load_gather(sw_sb, [curv, srow])
                mix = 1.0 / (1.0 + jnp.exp(-swv))
                om = 1.0 - mix
                for c in range(9):
                    col = jnp.full((16,), c, jnp.int32)
                    sv = plsc.load_gather(self_sb, [curv, srow, col])
                    val = om * sv + mix * accs[c]
                    dst = jnp.full((16,), c // 3, jnp.int32)
                    dcol = jnp.full((16,), c % 3, jnp.int32)
                    plsc.store_scatter(osb, [dst, srow, dcol], val)

                @pl.when(t + 4 < ng)
                def _():
                    fire(2 * t + 8, s0)
                    fire(2 * t + 9, s1)
            return carry2

        lax.fori_loop(0, ng // 4, round_body, 0)
        for j in range(3):
            pltpu.sync_copy(osb.at[j], out_hbm.at[j, pl.ds(base, SB)])

        @pl.when(sb + 1 < nsb)
        def _():
            wait_stage(1 - cur)
        return carry

    lax.fori_loop(0, nsb, sb_body, 0)


def _final_body(te_ref, av2_ref, ltt_ref, wv_ref, pd_ref, out_ref):
    te = te_ref[...]
    av2 = av2_ref[...]
    ltt = ltt_ref[...]
    wv3 = wv_ref[...]
    cidx = lax.broadcasted_iota(jnp.int32, av2.shape, 1)
    seas = cidx < 3
    wa = jnp.where(seas, wv3[0], 0.0)
    wr = jnp.where(seas, wv3[1], 0.0)
    wi = jnp.where(seas, wv3[2], 0.0)

    h2 = wr * wr + wi * wi
    inv = wa * lax.rsqrt(jnp.maximum(h2, 1e-30))
    a16 = inv * wr
    b16 = inv * wi
    ltv = av2 * ltt

    rr = lax.broadcasted_iota(jnp.int32, (16, T), 0)
    zero = jnp.zeros((16, T), jnp.float32)
    b_sin = zero
    b_cos = zero
    for i, f in enumerate(_SEASONAL_FREQS):
        arg = (_TWO_PI * f) * te
        b_sin = jnp.where(rr == i, jnp.sin(arg), b_sin)
        b_cos = jnp.where(rr == i, jnp.cos(arg), b_cos)
    b_lt = zero
    pdv = pd_ref[...]
    for j in range(2):
        arg = (_TWO_PI * te) / pdv[:, j:j + 1]
        b_lt = jnp.where(rr == j, jnp.sin(arg), b_lt)
        b_lt = jnp.where(rr == 2 + j, jnp.cos(arg), b_lt)
    b_poly = jnp.where(rr == 4, 1.0, jnp.where(rr == 5, te, 0.0))

    out_ref[...] = (_dot(av2, b_poly) + _dot(a16, b_sin) + _dot(b16, b_cos)
                    + _dot(ltv, b_lt))


def kernel(time_vector, linear_trend, constant_offset, seasonal_amplitudes,
           seasonal_phases, longterm_amplitudes, longterm_phases,
           longterm_periods, spatial_weights, neighbor_weights,
           neighbor_indices):
    f32 = jnp.float32
    n = linear_trend.shape[0]
    grid_n = (n + BN - 1) // BN

    av = jnp.concatenate(
        [seasonal_amplitudes, seasonal_phases, seasonal_phases,
         jnp.zeros((n, 7), f32)], axis=1)
    av2 = jnp.concatenate(
        [longterm_amplitudes, longterm_amplitudes,
         constant_offset[:, None], linear_trend[:, None],
         jnp.zeros((n, 10), f32)], axis=1)
    av3 = jnp.concatenate(
        [longterm_phases, longterm_phases, jnp.zeros((n, 12), f32)], axis=1)

    nw_rows = n * TBL // 128
    grid_p = (nw_rows + BN - 1) // BN
    wide_spec = dict(
        grid=(grid_p,),
        in_specs=[pl.BlockSpec((BN, 128), lambda i: (i, 0))],
        out_specs=pl.BlockSpec((BN, 128), lambda i: (i, 0)),
        out_shape=jax.ShapeDtypeStruct((nw_rows, 128), f32),
    )
    table = pl.pallas_call(_prep_body, **wide_spec)(
        av.reshape(nw_rows, 128)).reshape(n, TBL)
    ltt = pl.pallas_call(_prep_lt_body, **wide_spec)(
        av3.reshape(nw_rows, 128)).reshape(n, TBL)

    wv = pl.kernel(
        _sc_smooth_body,
        out_type=jax.ShapeDtypeStruct((3, n, TBL), f32),
        mesh=plsc.VectorSubcoreMesh(core_axis_name="c", subcore_axis_name="s",
                                    num_cores=2, num_subcores=16),
        compiler_params=pltpu.CompilerParams(needs_layout_passes=False,
                                             use_tc_tiling_on_sc=False),
        scratch_types=[
            pltpu.VMEM((2, SB, K), jnp.int32),
            pltpu.VMEM((2, SB, K), f32),
            pltpu.VMEM((2, SB, TBL), f32),
            pltpu.VMEM((2, SB), f32),
            pltpu.VMEM((SB * K,), jnp.int32),
            pltpu.VMEM((8, 128, TBL), f32),
            pltpu.VMEM((3, SB, TBL), f32),
            pltpu.SemaphoreType.DMA((12,)),
        ],
    )(table, neighbor_indices, neighbor_weights, spatial_weights)

    te2 = time_vector.reshape(1, T)
    pd2 = longterm_periods.reshape(1, 2)
    out = pl.pallas_call(
        _final_body,
        grid=(grid_n,),
        in_specs=[
            pl.BlockSpec((1, T), lambda i: (0, 0)),
            pl.BlockSpec((BN, TBL), lambda i: (i, 0)),
            pl.BlockSpec((BN, TBL), lambda i: (i, 0)),
            pl.BlockSpec((3, BN, TBL), lambda i: (0, i, 0)),
            pl.BlockSpec((1, 2), lambda i: (0, 0)),
        ],
        out_specs=pl.BlockSpec((BN, T), lambda i: (i, 0)),
        out_shape=jax.ShapeDtypeStruct((n, T), f32),
    )(te2, av2, ltt, wv, pd2)
    return out


# speedup vs baseline: 172.6378x; 1.0459x over previous
import math

import jax
import jax.numpy as jnp
from jax import lax
from jax.experimental import pallas as pl
from jax.experimental.pallas import tpu as pltpu
from jax.experimental.pallas import tpu_sc as plsc

K = 16
TBL = 16
T = 128
NW = 32
SB = 640
G = 16
BN = 2048

_SEASONAL_FREQS = (4.0, 2.0, 1.0)
_TWO_PI = 2.0 * math.pi


def _dot(a, b):
    return jnp.dot(a.astype(jnp.bfloat16), b.astype(jnp.bfloat16),
                   preferred_element_type=jnp.float32)


def _prep_body(av_ref, out_ref):
    xw = av_ref[...]
    c = lax.broadcasted_iota(jnp.int32, xw.shape, 1) % 16
    out_ref[...] = jnp.where(c < 3, xw,
                             jnp.where(c < 6, jnp.cos(xw),
                                       jnp.where(c < 9, jnp.sin(xw), 0.0)))


def _prep_lt_body(av3_ref, out_ref):
    xw = av3_ref[...]
    c = lax.broadcasted_iota(jnp.int32, xw.shape, 1) % 16
    out_ref[...] = jnp.where(c < 2, jnp.cos(xw),
                             jnp.where(c < 4, jnp.sin(xw), 0.0))


def _sc_smooth_body(table_hbm, idx_hbm, w_hbm, sw_hbm, out_hbm,
                    idx_sb, w_sb, self_sb, sw_sb, idx_flat, rows_v, osb, sem):
    n = out_hbm.shape[1]
    s_w = (n + NW - 1) // NW
    s_w = s_w + (SB - s_w % SB) % SB
    nsb = s_w // SB
    ng = SB // G
    clamp = n - SB
    wid = lax.axis_index("s") * 2 + lax.axis_index("c")
    lane = lax.iota(jnp.int32, 16)
    hi = lane >> 3

    def stage(sb, buf):
        base = pl.multiple_of(
            jnp.minimum(wid * s_w + sb * SB, clamp), 8)
        pltpu.async_copy(idx_hbm.at[pl.ds(base, SB)], idx_sb.at[buf],
                         sem.at[16])
        pltpu.async_copy(w_hbm.at[pl.ds(base, SB)], w_sb.at[buf],
                         sem.at[17])
        pltpu.async_copy(sw_hbm.at[pl.ds(base, SB)], sw_sb.at[buf],
                         sem.at[18])

    def wait_stage(buf):
        pltpu.make_async_copy(idx_hbm.at[pl.ds(0, SB)], idx_sb.at[buf],
                              sem.at[16]).wait()
        pltpu.make_async_copy(w_hbm.at[pl.ds(0, SB)], w_sb.at[buf],
                              sem.at[17]).wait()
        pltpu.make_async_copy(sw_hbm.at[pl.ds(0, SB)], sw_sb.at[buf],
                              sem.at[18]).wait()

    def fire(half, slot):
        pltpu.async_copy(table_hbm.at[idx_flat.at[pl.ds(half * 128, 128)]],
                         rows_v.at[slot], sem.at[slot])

    def wait_slot(slot):
        pltpu.make_async_copy(table_hbm.at[pl.ds(0, 128)], rows_v.at[slot],
                              sem.at[slot]).wait()

    stage(0, 0)
    wait_stage(0)

    def sb_body(sb, carry):
        cur = lax.rem(sb, 2)
        base = pl.multiple_of(
            jnp.minimum(wid * s_w + sb * SB, clamp), 8)

        @pl.when(sb + 1 < nsb)
        def _():
            stage(sb + 1, 1 - cur)

        pltpu.sync_copy(table_hbm.at[pl.ds(base, SB)], self_sb)

        def repack_body(s, carry2):
            idx_flat[pl.ds(s * K, K)] = idx_sb[cur, s]
            return carry2

        lax.fori_loop(0, SB, repack_body, 0)

        for j in range(16):
            fire(j, j)

        def round_body(r, carry2):
            for gi in range(8):
                t = 8 * r + gi
                s0, s1 = 2 * gi, 2 * gi + 1
                wait_slot(s0)
                wait_slot(s1)
                srow = t * G + lane
                curv = jnp.full((16,), cur, jnp.int32)
                bsel = s0 + hi
                lane16 = (lane & 7) * K

                def k_body(k, accs):
                    kv = jnp.full((16,), k, jnp.int32)
                    wk = plsc.load_gather(w_sb, [curv, srow, kv])
                    lo = lane16 + k
                    return tuple(
                        accs[c] + wk * plsc.load_gather(
                            rows_v, [bsel, lo, jnp.full((16,), c, jnp.int32)])
                        for c in range(9))

                accs = lax.fori_loop(
                    0, K, k_body,
                    tuple(jnp.zeros((16,), jnp.float32) for _ in range(9)))
                swv = plsc.load_gather(sw_sb, [curv, srow])
                mix = 1.0 / (1.0 + jnp.exp(-swv))
                om = 1.0 - mix
                for c in range(9):
                    col = jnp.full((16,), c, jnp.int32)
                    sv = plsc.load_gather(self_sb, [srow, col])
                    val = om * sv + mix * accs[c]
                    dst = jnp.full((16,), c // 3, jnp.int32)
                    dcol = jnp.full((16,), c % 3, jnp.int32)
                    plsc.store_scatter(osb, [dst, srow, dcol], val)

                @pl.when(t + 8 < ng)
                def _():
                    fire(2 * t + 16, s0)
                    fire(2 * t + 17, s1)
            return carry2

        lax.fori_loop(0, ng // 8, round_body, 0)
        for j in range(3):
            pltpu.sync_copy(osb.at[j], out_hbm.at[j, pl.ds(base, SB)])

        @pl.when(sb + 1 < nsb)
        def _():
            wait_stage(1 - cur)
        return carry

    lax.fori_loop(0, nsb, sb_body, 0)


def _final_body(te_ref, av2_ref, ltt_ref, wv_ref, pd_ref, out_ref):
    te = te_ref[...]
    av2 = av2_ref[...]
    ltt = ltt_ref[...]
    wv3 = wv_ref[...]
    cidx = lax.broadcasted_iota(jnp.int32, av2.shape, 1)
    seas = cidx < 3
    wa = jnp.where(seas, wv3[0], 0.0)
    wr = jnp.where(seas, wv3[1], 0.0)
    wi = jnp.where(seas, wv3[2], 0.0)

    h2 = wr * wr + wi * wi
    inv = wa * lax.rsqrt(jnp.maximum(h2, 1e-30))
    a16 = inv * wr
    b16 = inv * wi
    ltv = av2 * ltt

    rr = lax.broadcasted_iota(jnp.int32, (16, T), 0)
    zero = jnp.zeros((16, T), jnp.float32)
    b_sin = zero
    b_cos = zero
    for i, f in enumerate(_SEASONAL_FREQS):
        arg = (_TWO_PI * f) * te
        b_sin = jnp.where(rr == i, jnp.sin(arg), b_sin)
        b_cos = jnp.where(rr == i, jnp.cos(arg), b_cos)
    b_lt = zero
    pdv = pd_ref[...]
    for j in range(2):
        arg = (_TWO_PI * te) / pdv[:, j:j + 1]
        b_lt = jnp.where(rr == j, jnp.sin(arg), b_lt)
        b_lt = jnp.where(rr == 2 + j, jnp.cos(arg), b_lt)
    b_poly = jnp.where(rr == 4, 1.0, jnp.where(rr == 5, te, 0.0))

    out_ref[...] = (_dot(av2, b_poly) + _dot(a16, b_sin) + _dot(b16, b_cos)
                    + _dot(ltv, b_lt))


def kernel(time_vector, linear_trend, constant_offset, seasonal_amplitudes,
           seasonal_phases, longterm_amplitudes, longterm_phases,
           longterm_periods, spatial_weights, neighbor_weights,
           neighbor_indices):
    f32 = jnp.float32
    n = linear_trend.shape[0]
    grid_n = (n + BN - 1) // BN

    av = jnp.concatenate(
        [seasonal_amplitudes, seasonal_phases, seasonal_phases,
         jnp.zeros((n, 7), f32)], axis=1)
    av2 = jnp.concatenate(
        [longterm_amplitudes, longterm_amplitudes,
         constant_offset[:, None], linear_trend[:, None],
         jnp.zeros((n, 10), f32)], axis=1)
    av3 = jnp.concatenate(
        [longterm_phases, longterm_phases, jnp.zeros((n, 12), f32)], axis=1)

    nw_rows = n * TBL // 128
    grid_p = (nw_rows + BN - 1) // BN
    wide_spec = dict(
        grid=(grid_p,),
        in_specs=[pl.BlockSpec((BN, 128), lambda i: (i, 0))],
        out_specs=pl.BlockSpec((BN, 128), lambda i: (i, 0)),
        out_shape=jax.ShapeDtypeStruct((nw_rows, 128), f32),
    )
    table = pl.pallas_call(_prep_body, **wide_spec)(
        av.reshape(nw_rows, 128)).reshape(n, TBL)
    ltt = pl.pallas_call(_prep_lt_body, **wide_spec)(
        av3.reshape(nw_rows, 128)).reshape(n, TBL)

    wv = pl.kernel(
        _sc_smooth_body,
        out_type=jax.ShapeDtypeStruct((3, n, TBL), f32),
        mesh=plsc.VectorSubcoreMesh(core_axis_name="c", subcore_axis_name="s",
                                    num_cores=2, num_subcores=16),
        compiler_params=pltpu.CompilerParams(needs_layout_passes=False,
                                             use_tc_tiling_on_sc=False),
        scratch_types=[
            pltpu.VMEM((2, SB, K), jnp.int32),
            pltpu.VMEM((2, SB, K), f32),
            pltpu.VMEM((SB, TBL), f32),
            pltpu.VMEM((2, SB), f32),
            pltpu.VMEM((SB * K,), jnp.int32),
            pltpu.VMEM((16, 128, TBL), f32),
            pltpu.VMEM((3, SB, TBL), f32),
            pltpu.SemaphoreType.DMA((20,)),
        ],
    )(table, neighbor_indices, neighbor_weights, spatial_weights)

    te2 = time_vector.reshape(1, T)
    pd2 = longterm_periods.reshape(1, 2)
    out = pl.pallas_call(
        _final_body,
        grid=(grid_n,),
        in_specs=[
            pl.BlockSpec((1, T), lambda i: (0, 0)),
            pl.BlockSpec((BN, TBL), lambda i: (i, 0)),
            pl.BlockSpec((BN, TBL), lambda i: (i, 0)),
            pl.BlockSpec((3, BN, TBL), lambda i: (0, i, 0)),
            pl.BlockSpec((1, 2), lambda i: (0, 0)),
        ],
        out_specs=pl.BlockSpec((BN, T), lambda i: (i, 0)),
        out_shape=jax.ShapeDtypeStruct((n, T), f32),
    )(te2, av2, ltt, wv, pd2)
    return out
